# baseline (speedup 1.0000x reference)
import functools

import jax
import jax.numpy as jnp
from jax import lax
from jax.experimental import pallas as pl
from jax.experimental.pallas import tpu as pltpu
from jax.experimental.pallas import tpu_sc as plsc

_GROUP = 16
_LANES = 16


def _oddeven_merge_sort_pairs(n):
    pairs = []

    def merge(lo, m, r):
        step = r * 2
        if step < m:
            merge(lo, m, step)
            merge(lo + r, m, step)
            for i in range(lo + r, lo + m - r, step):
                pairs.append((i, i + r))
        else:
            pairs.append((lo, lo + r))

    def sort(lo, m):
        if m > 1:
            half = m // 2
            sort(lo, half)
            sort(lo + half, half)
            merge(lo, m, 1)

    sort(0, n)
    return pairs


_PAIRS = tuple(_oddeven_merge_sort_pairs(_GROUP))


@functools.lru_cache(maxsize=None)
def _make_sc_sort(n_rows, f, chunk_rows, nbuf):
    groups_per_row = f // _GROUP
    blocks_per_row = groups_per_row // _LANES
    info = plsc.get_sparse_core_info()
    num_workers = info.num_cores * info.num_subcores
    rows_per_worker = n_rows // num_workers
    chunks = rows_per_worker // chunk_rows
    chunk_words = chunk_rows * f
    assert chunks % nbuf == 0

    mesh = plsc.VectorSubcoreMesh(core_axis_name="c", subcore_axis_name="s")

    @functools.partial(
        pl.kernel,
        out_type=jax.ShapeDtypeStruct((n_rows, f), jnp.float32),
        mesh=mesh,
        scratch_types=(
            [pltpu.VMEM((chunk_rows, f), jnp.float32)] * (2 * nbuf)
            + [pltpu.SemaphoreType.DMA] * (2 * nbuf)
        ),
    )
    def sc_sort(x_hbm, out_hbm, *bufs):
        inb = bufs[:nbuf]
        otb = bufs[nbuf : 2 * nbuf]
        isem = bufs[2 * nbuf : 3 * nbuf]
        osem = bufs[3 * nbuf :]
        wid = lax.axis_index("s") * info.num_cores + lax.axis_index("c")
        worker_row = wid * rows_per_worker

        def load(c, b):
            return pltpu.make_async_copy(
                x_hbm.at[pl.ds(worker_row + c * chunk_rows, chunk_rows)],
                inb[b],
                isem[b],
            )

        def store(c, b):
            return pltpu.make_async_copy(
                otb[b],
                out_hbm.at[pl.ds(worker_row + c * chunk_rows, chunk_rows)],
                osem[b],
            )

        def sort_chunk(b):
            src = inb[b]
            dst = otb[b]

            def row_body(r, _):
                for j in range(blocks_per_row):
                    base = j * _LANES
                    v = [
                        src[r, pl.ds(base + i * groups_per_row, _LANES)]
                        for i in range(_GROUP)
                    ]
                    for a, bb in _PAIRS:
                        lo = jnp.minimum(v[a], v[bb])
                        hi = jnp.maximum(v[a], v[bb])
                        v[a] = lo
                        v[bb] = hi
                    for i in range(_GROUP):
                        dst[r, pl.ds(base + i * groups_per_row, _LANES)] = v[i]
                return 0

            lax.fori_loop(0, chunk_rows, row_body, 0)

        for b in range(nbuf):
            load(b, b).start()

        def it_body(it, _):
            for b in range(nbuf):
                c = it * nbuf + b
                load(c, b).wait()

                @pl.when(it > 0)
                def _():
                    store(c, b).wait()

                sort_chunk(b)
                store(c, b).start()

                @pl.when(c + nbuf < chunks)
                def _():
                    load(c + nbuf, b).start()

            return 0

        lax.fori_loop(0, chunks // nbuf, it_body, 0)
        for b in range(nbuf):
            store(chunks - nbuf + b, b).wait()

    return sc_sort


@functools.lru_cache(maxsize=None)
def _make_tc_sort(n_rows, n_skip, f, block_rows):
    groups_per_row = f // _GROUP
    skip_blocks = n_skip // block_rows

    def tc_body(x_ref, o_ref):
        v = [
            x_ref[:, i * groups_per_row : (i + 1) * groups_per_row]
            for i in range(_GROUP)
        ]
        for a, b in _PAIRS:
            lo = jnp.minimum(v[a], v[b])
            hi = jnp.maximum(v[a], v[b])
            v[a] = lo
            v[b] = hi
        for i in range(_GROUP):
            o_ref[:, i * groups_per_row : (i + 1) * groups_per_row] = v[i]

    return pl.pallas_call(
        tc_body,
        out_shape=jax.ShapeDtypeStruct((n_rows, f), jnp.float32),
        grid=((n_rows - n_skip) // block_rows,),
        in_specs=[pl.BlockSpec((block_rows, f), lambda i: (i + skip_blocks, 0))],
        out_specs=pl.BlockSpec((block_rows, f), lambda i: (i + skip_blocks, 0)),
    )


def kernel(x):
    n, f = x.shape
    n_sc = 12288
    sc_sort = _make_sc_sort(n_sc, f, 8, 3)
    tc_sort = _make_tc_sort(n, n_sc, f, 256)
    out_sc = sc_sort(x)
    out_tc = tc_sort(x)
    return jax.lax.dynamic_update_slice(out_tc, out_sc, (0, 0))


# speedup vs baseline: 1.0806x; 1.0806x over previous
import functools

import jax
import jax.numpy as jnp
from jax import lax
from jax.experimental import pallas as pl
from jax.experimental.pallas import tpu as pltpu
from jax.experimental.pallas import tpu_sc as plsc

_GROUP = 16
_LANES = 16


def _oddeven_merge_sort_pairs(n):
    pairs = []

    def merge(lo, m, r):
        step = r * 2
        if step < m:
            merge(lo, m, step)
            merge(lo + r, m, step)
            for i in range(lo + r, lo + m - r, step):
                pairs.append((i, i + r))
        else:
            pairs.append((lo, lo + r))

    def sort(lo, m):
        if m > 1:
            half = m // 2
            sort(lo, half)
            sort(lo + half, half)
            merge(lo, m, 1)

    sort(0, n)
    return pairs


_PAIRS = tuple(_oddeven_merge_sort_pairs(_GROUP))


@functools.lru_cache(maxsize=None)
def _make_sc_sort(n_rows, f, chunk_rows, nbuf):
    groups_per_row = f // _GROUP
    blocks_per_row = groups_per_row // _LANES
    info = plsc.get_sparse_core_info()
    num_workers = info.num_cores * info.num_subcores
    rows_per_worker = n_rows // num_workers
    chunks = rows_per_worker // chunk_rows
    chunk_words = chunk_rows * f
    assert chunks % nbuf == 0

    mesh = plsc.VectorSubcoreMesh(core_axis_name="c", subcore_axis_name="s")

    @functools.partial(
        pl.kernel,
        out_type=jax.ShapeDtypeStruct((n_rows, f), jnp.float32),
        mesh=mesh,
        scratch_types=(
            [pltpu.VMEM((chunk_rows, f), jnp.float32)] * (2 * nbuf)
            + [pltpu.SemaphoreType.DMA] * (2 * nbuf)
        ),
    )
    def sc_sort(x_hbm, out_hbm, *bufs):
        inb = bufs[:nbuf]
        otb = bufs[nbuf : 2 * nbuf]
        isem = bufs[2 * nbuf : 3 * nbuf]
        osem = bufs[3 * nbuf :]
        wid = lax.axis_index("s") * info.num_cores + lax.axis_index("c")
        worker_row = wid * rows_per_worker

        def load(c, b):
            return pltpu.make_async_copy(
                x_hbm.at[pl.ds(worker_row + c * chunk_rows, chunk_rows)],
                inb[b],
                isem[b],
            )

        def store(c, b):
            return pltpu.make_async_copy(
                otb[b],
                out_hbm.at[pl.ds(worker_row + c * chunk_rows, chunk_rows)],
                osem[b],
            )

        def sort_chunk(b):
            src = inb[b]
            dst = otb[b]

            def row_body(r, _):
                for j in range(blocks_per_row):
                    base = j * _LANES
                    v = [
                        src[r, pl.ds(base + i * groups_per_row, _LANES)]
                        for i in range(_GROUP)
                    ]
                    for a, bb in _PAIRS:
                        lo = jnp.minimum(v[a], v[bb])
                        hi = jnp.maximum(v[a], v[bb])
                        v[a] = lo
                        v[bb] = hi
                    for i in range(_GROUP):
                        dst[r, pl.ds(base + i * groups_per_row, _LANES)] = v[i]
                return 0

            lax.fori_loop(0, chunk_rows, row_body, 0)

        for b in range(nbuf):
            load(b, b).start()

        def it_body(it, _):
            for b in range(nbuf):
                c = it * nbuf + b
                load(c, b).wait()

                @pl.when(it > 0)
                def _():
                    store(c, b).wait()

                sort_chunk(b)
                store(c, b).start()

                @pl.when(c + nbuf < chunks)
                def _():
                    load(c + nbuf, b).start()

            return 0

        lax.fori_loop(0, chunks // nbuf, it_body, 0)
        for b in range(nbuf):
            store(chunks - nbuf + b, b).wait()

    return sc_sort


@functools.lru_cache(maxsize=None)
def _make_tc_sort(n_rows, n_skip, f, block_rows, nbuf=2):
    groups_per_row = f // _GROUP
    chunks = (n_rows - n_skip) // block_rows
    assert chunks % nbuf == 0

    def net(src, dst):
        v = [
            src[:, i * groups_per_row : (i + 1) * groups_per_row]
            for i in range(_GROUP)
        ]
        for a, b in _PAIRS:
            lo = jnp.minimum(v[a], v[b])
            hi = jnp.maximum(v[a], v[b])
            v[a] = lo
            v[b] = hi
        for i in range(_GROUP):
            dst[:, i * groups_per_row : (i + 1) * groups_per_row] = v[i]

    def tc_body(x_hbm, o_hbm, *bufs):
        inb = bufs[:nbuf]
        otb = bufs[nbuf : 2 * nbuf]
        isem = bufs[2 * nbuf : 3 * nbuf]
        osem = bufs[3 * nbuf :]

        def load(c, b):
            return pltpu.make_async_copy(
                x_hbm.at[pl.ds(n_skip + c * block_rows, block_rows)],
                inb[b],
                isem[b],
            )

        def store(c, b):
            return pltpu.make_async_copy(
                otb[b],
                o_hbm.at[pl.ds(n_skip + c * block_rows, block_rows)],
                osem[b],
            )

        for b in range(nbuf):
            load(b, b).start()

        def it_body(it, _):
            for b in range(nbuf):
                c = it * nbuf + b
                load(c, b).wait()

                @pl.when(it > 0)
                def _():
                    store(c, b).wait()

                net(inb[b], otb[b])
                store(c, b).start()

                @pl.when(c + nbuf < chunks)
                def _():
                    load(c + nbuf, b).start()

            return 0

        lax.fori_loop(0, chunks // nbuf, it_body, 0)
        for b in range(nbuf):
            store(chunks - nbuf + b, b).wait()

    return pl.pallas_call(
        tc_body,
        out_shape=jax.ShapeDtypeStruct((n_rows, f), jnp.float32),
        in_specs=[pl.BlockSpec(memory_space=pl.ANY)],
        out_specs=pl.BlockSpec(memory_space=pl.ANY),
        scratch_shapes=(
            [pltpu.VMEM((block_rows, f), jnp.float32)] * (2 * nbuf)
            + [pltpu.SemaphoreType.DMA] * (2 * nbuf)
        ),
    )


def kernel(x):
    n, f = x.shape
    n_sc = 12288
    sc_sort = _make_sc_sort(n_sc, f, 8, 2)
    tc_sort = _make_tc_sort(n, n_sc, f, 256, 4)
    out_sc = sc_sort(x)
    out_tc = tc_sort(x)
    return jax.lax.dynamic_update_slice(out_tc, out_sc, (0, 0))
